# baseline (speedup 1.0000x reference)
import functools

import jax
import jax.numpy as jnp
from jax import lax
from jax.experimental import pallas as pl
from jax.experimental.pallas import tpu as pltpu
from jax.experimental.pallas import tpu_sc as plsc

_NUM_IN = 20
_NUM_OUT = 64
_BATCH = 16384
_NW = 32
_BPW = _BATCH // _NW
_CHUNK = 128
_NCHUNK = _BPW // _CHUNK
_GRP = 16
_NGRP = _CHUNK // _GRP


def _sc_body(x_hbm, lut_hbm, out_hbm,
             xv, idx0, idx1, idx2, idx3, r0, r1, r2, r3,
             s0, s1, s2, s3):
    wid = lax.axis_index("s") * 2 + lax.axis_index("c")
    base = wid * _BPW
    pltpu.sync_copy(x_hbm.at[pl.ds(base * _NUM_IN, _BPW * _NUM_IN)], xv)

    iota = lax.iota(jnp.int32, _GRP)
    idx_refs = (idx0, idx1, idx2, idx3)
    row_refs = (r0, r1, r2, r3)
    sems = (s0, s1, s2, s3)
    descs = []
    for j in range(_NCHUNK):
        def grp(k, carry, j=j):
            row_base = ((j * _NGRP + k) * _GRP + iota) * _NUM_IN
            acc = jnp.zeros((_GRP,), jnp.int32)
            for i in range(_NUM_IN):
                v = plsc.load_gather(xv, [row_base + i])
                acc = acc + jnp.where(v > 0.0, jnp.int32(1 << i),
                                      jnp.int32(0))
            idx_refs[j][pl.ds(k * _GRP, _GRP)] = acc
            return carry
        lax.fori_loop(0, _NGRP, grp, 0)
        descs.append(
            pltpu.async_copy(lut_hbm.at[idx_refs[j]], row_refs[j], sems[j]))
    for j in range(_NCHUNK):
        descs[j].wait()
        pltpu.sync_copy(row_refs[j],
                        out_hbm.at[pl.ds(base + j * _CHUNK, _CHUNK)])


@functools.partial(jax.jit, static_argnames=())
def kernel(x, lut):
    mesh = plsc.VectorSubcoreMesh(core_axis_name="c", subcore_axis_name="s")
    run = pl.kernel(
        _sc_body,
        out_type=jax.ShapeDtypeStruct((_BATCH, _NUM_OUT), jnp.float32),
        mesh=mesh,
        scratch_types=(
            [pltpu.VMEM((_BPW * _NUM_IN,), jnp.float32)]
            + [pltpu.VMEM((_CHUNK,), jnp.int32) for _ in range(_NCHUNK)]
            + [pltpu.VMEM((_CHUNK, _NUM_OUT), jnp.float32)
               for _ in range(_NCHUNK)]
            + [pltpu.SemaphoreType.DMA for _ in range(_NCHUNK)]
        ),
        compiler_params=pltpu.CompilerParams(
            needs_layout_passes=False, use_tc_tiling_on_sc=False),
    )
    return run(x.reshape(-1), lut)


# speedup vs baseline: 6.1823x; 6.1823x over previous
import functools

import jax
import jax.numpy as jnp
from jax import lax
from jax.experimental import pallas as pl
from jax.experimental.pallas import tpu as pltpu
from jax.experimental.pallas import tpu_sc as plsc

_NUM_IN = 20
_NUM_OUT = 64
_BATCH = 16384
_NW = 32
_BPW = _BATCH // _NW
_GRP = 16
_NGRP = _BPW // _GRP
_ELEMS = _BPW * _NUM_OUT
_ROW = 128
_NROW = _ELEMS // _ROW


def _sc_body(x_hbm, lut_hbm, out_hbm, xv, idxl, dst, sem):
    wid = lax.axis_index("s") * 2 + lax.axis_index("c")
    base = wid * _BPW
    pltpu.sync_copy(x_hbm.at[pl.ds(base * _NUM_IN, _BPW * _NUM_IN)], xv)

    iota = lax.iota(jnp.int32, _GRP)
    lane64 = iota * _NUM_OUT

    def grp(g, carry):
        row_base = (g * _GRP + iota) * _NUM_IN
        acc = jnp.zeros((_GRP,), jnp.int32)
        for i in range(_NUM_IN):
            v = plsc.load_gather(xv, [row_base + i])
            acc = acc + jnp.where(v > 0.0, jnp.int32(1 << i), jnp.int32(0))
        addr0 = ((acc >> 7) << 10) + (acc & 127)
        pos_base = g * (_GRP * _NUM_OUT) + lane64
        for c in range(_NUM_OUT):
            off = (c // 8) * 8388608 + (c % 8) * 128
            pos = pos_base + c
            plsc.store_scatter(idxl, [pos >> 7, pos & 127], addr0 + off)
        return carry

    lax.fori_loop(0, _NGRP, grp, 0)

    def fire(j, carry):
        pltpu.async_copy(lut_hbm.at[idxl.at[j]],
                         dst.at[pl.ds(j * _ROW, _ROW)], sem)
        return carry

    lax.fori_loop(0, _NROW, fire, 0)
    pltpu.make_async_copy(
        lut_hbm.at[pl.ds(0, _ELEMS)], dst, sem).wait()
    pltpu.sync_copy(dst, out_hbm.at[pl.ds(base * _NUM_OUT, _ELEMS)])


@functools.partial(jax.jit, static_argnames=())
def kernel(x, lut):
    mesh = plsc.VectorSubcoreMesh(core_axis_name="c", subcore_axis_name="s")
    run = pl.kernel(
        _sc_body,
        out_type=jax.ShapeDtypeStruct((_BATCH * _NUM_OUT,), jnp.float32),
        mesh=mesh,
        scratch_types=(
            pltpu.VMEM((_BPW * _NUM_IN,), jnp.float32),
            pltpu.VMEM((_NROW, _ROW), jnp.int32),
            pltpu.VMEM((_ELEMS,), jnp.float32),
            pltpu.SemaphoreType.DMA,
        ),
        compiler_params=pltpu.CompilerParams(
            needs_layout_passes=False, use_tc_tiling_on_sc=False),
    )
    lut_flat = (
        lut.T.reshape(8, 8, 8192, 128).transpose(0, 2, 1, 3).reshape(-1)
    )
    out = run(x.reshape(-1), lut_flat)
    return out.reshape(_BATCH, _NUM_OUT)


# speedup vs baseline: 8.5856x; 1.3887x over previous
import functools

import jax
import jax.numpy as jnp
from jax import lax
from jax.experimental import pallas as pl
from jax.experimental.pallas import tpu as pltpu
from jax.experimental.pallas import tpu_sc as plsc

_NUM_IN = 20
_NUM_OUT = 64
_BATCH = 16384
_NW = 32
_BPW = _BATCH // _NW
_GRP = 16
_NGRP = _BPW // _GRP
_ELEMS = _BPW * _NUM_OUT
_ROW = 128
_NROW = _ELEMS // _ROW


def _sc_body(x_hbm, lut_hbm, out_hbm, xv, idxl, dst, sem):
    wid = lax.axis_index("s") * 2 + lax.axis_index("c")
    base = wid * _BPW
    pltpu.sync_copy(x_hbm.at[pl.ds(base * _NUM_IN, _BPW * _NUM_IN)], xv)

    iota = lax.iota(jnp.int32, _GRP)

    def grp(g, carry):
        row_base = (g * _GRP + iota) * _NUM_IN
        acc = jnp.zeros((_GRP,), jnp.int32)
        for i in range(_NUM_IN):
            v = plsc.load_gather(xv, [row_base + i])
            acc = acc + jnp.where(v > 0.0, jnp.int32(1 << i), jnp.int32(0))
        addr0 = ((acc >> 7) << 10) + (acc & 127)
        q = g >> 3
        col = (g & 7) * _GRP
        for c in range(_NUM_OUT):
            off = (c // 8) * 8388608 + (c % 8) * 128
            row = (c // 8) * 32 + q * 8 + (c % 8)
            idxl[row, pl.ds(col, _GRP)] = addr0 + off

        @pl.when((g & 7) == 7)
        def _fire():
            for cb8 in range(8):
                for s in range(8):
                    j = cb8 * 32 + q * 8 + s
                    pltpu.async_copy(lut_hbm.at[idxl.at[j]],
                                     dst.at[pl.ds(j * _ROW, _ROW)], sem)

        return carry

    lax.fori_loop(0, _NGRP, grp, 0)
    pltpu.make_async_copy(lut_hbm.at[pl.ds(0, _ELEMS)], dst, sem).wait()
    for cb8 in range(8):
        pltpu.sync_copy(
            dst.at[pl.ds(cb8 * 4096, 4096)],
            out_hbm.at[pl.ds(cb8 * 131072 + wid * 4096, 4096)])


@functools.partial(jax.jit, static_argnames=())
def kernel(x, lut):
    mesh = plsc.VectorSubcoreMesh(core_axis_name="c", subcore_axis_name="s")
    run = pl.kernel(
        _sc_body,
        out_type=jax.ShapeDtypeStruct((_BATCH * _NUM_OUT,), jnp.float32),
        mesh=mesh,
        scratch_types=(
            pltpu.VMEM((_BPW * _NUM_IN,), jnp.float32),
            pltpu.VMEM((_NROW, _ROW), jnp.int32),
            pltpu.VMEM((_ELEMS,), jnp.float32),
            pltpu.SemaphoreType.DMA,
        ),
        compiler_params=pltpu.CompilerParams(
            needs_layout_passes=False, use_tc_tiling_on_sc=False),
    )
    lut_flat = (
        lut.T.reshape(8, 8, 8192, 128).transpose(0, 2, 1, 3).reshape(-1)
    )
    out = run(x.reshape(-1), lut_flat)
    return out.reshape(8, 128, 8, 128).transpose(0, 2, 1, 3).reshape(
        _NUM_OUT, _BATCH).T
